# Initial kernel scaffold; baseline (speedup 1.0000x reference)
import jax
import jax.numpy as jnp
from jax.experimental import pallas as pl


_KH = 14


def _pool_body(x_ref, o_ref):
    x = x_ref[0]
    y = jnp.maximum(x, 0.0)
    t = y[0::2]
    b = y[1::2]
    r = t + b
    r2 = t * t + b * b
    s = r[:, 0::2, :] + r[:, 1::2, :]
    s2 = r2[:, 0::2, :] + r2[:, 1::2, :]
    o_ref[0] = jnp.where(s > 0, s2 / jnp.where(s > 0, s, 1.0), 0.0)


def kernel(x):
    B, H, W, C = x.shape
    HO, WO = H // 2, W // 2
    grid = (B, HO // _KH)
    return pl.pallas_call(
        _pool_body,
        grid=grid,
        in_specs=[pl.BlockSpec((1, 2 * _KH, W, C), lambda b, i: (b, i, 0, 0))],
        out_specs=pl.BlockSpec((1, _KH, WO, C), lambda b, i: (b, i, 0, 0)),
        out_shape=jax.ShapeDtypeStruct((B, HO, WO, C), x.dtype),
    )(x)


# speedup vs baseline: 1.0029x; 1.0029x over previous
import jax
import jax.numpy as jnp
from jax.experimental import pallas as pl


_KH = 14


def _pool_body(xt_ref, xb_ref, o_ref):
    C = o_ref.shape[-1]
    t = jnp.maximum(xt_ref[0, :, 0, :, :], 0.0)
    b = jnp.maximum(xb_ref[0, :, 0, :, :], 0.0)
    r = t + b
    r2 = t * t + b * b
    s = r[:, :, :C] + r[:, :, C:]
    s2 = r2[:, :, :C] + r2[:, :, C:]
    o_ref[0] = jnp.where(s > 0, s2 / jnp.where(s > 0, s, 1.0), 0.0)


def kernel(x):
    B, H, W, C = x.shape
    HO, WO = H // 2, W // 2
    xv = x.reshape(B, HO, 2, WO, 2 * C)
    grid = (B, HO // _KH)
    blk = (1, _KH, 1, WO, 2 * C)

    def mk(h):
        return pl.BlockSpec(blk, lambda b, i: (b, i, h, 0, 0))

    return pl.pallas_call(
        _pool_body,
        grid=grid,
        in_specs=[mk(0), mk(1)],
        out_specs=pl.BlockSpec((1, _KH, WO, C), lambda b, i: (b, i, 0, 0)),
        out_shape=jax.ShapeDtypeStruct((B, HO, WO, C), x.dtype),
    )(xv, xv)


# speedup vs baseline: 1.1315x; 1.1282x over previous
import jax
import jax.numpy as jnp
from jax.experimental import pallas as pl


_KH = 14


def _pool_body(xt_ref, xb_ref, o_ref):
    KH, WO, C = o_ref.shape[1:]
    t = jnp.maximum(xt_ref[0, :, 0], 0.0)
    b = jnp.maximum(xb_ref[0, :, 0], 0.0)
    r = t + b
    r2 = t * t + b * b
    rw = r.reshape(KH, WO, 2, C)
    r2w = r2.reshape(KH, WO, 2, C)
    s = rw[:, :, 0, :] + rw[:, :, 1, :]
    s2 = r2w[:, :, 0, :] + r2w[:, :, 1, :]
    o_ref[0] = jnp.where(s > 0, s2 / jnp.where(s > 0, s, 1.0), 0.0)


def kernel(x):
    B, H, W, C = x.shape
    HO, WO = H // 2, W // 2
    xv = x.reshape(B, HO, 2, W, C)
    grid = (B, HO // _KH)
    blk = (1, _KH, 1, W, C)

    def mk(h):
        return pl.BlockSpec(blk, lambda b, i: (b, i, h, 0, 0))

    return pl.pallas_call(
        _pool_body,
        grid=grid,
        in_specs=[mk(0), mk(1)],
        out_specs=pl.BlockSpec((1, _KH, WO, C), lambda b, i: (b, i, 0, 0)),
        out_shape=jax.ShapeDtypeStruct((B, HO, WO, C), x.dtype),
    )(xv, xv)


# speedup vs baseline: 3.0130x; 2.6629x over previous
import jax
import jax.numpy as jnp
from jax.experimental import pallas as pl
from jax.experimental.pallas import tpu as pltpu


_KH = 14


def _pool_body(x_ref, o_ref):
    KH, WO, C = o_ref.shape[1:]
    y = jnp.maximum(x_ref[0], 0.0)
    yr = y.reshape(KH, 2, 2 * WO, C)
    t = yr[:, 0]
    b = yr[:, 1]
    r = t + b
    r2 = t * t + b * b
    z = r + pltpu.roll(r, 1, 1)
    z2 = r2 + pltpu.roll(r2, 1, 1)
    q = jnp.where(z > 0, z2 / jnp.where(z > 0, z, 1.0), 0.0)
    o_ref[0] = q.reshape(KH, WO, 2, C)[:, :, 1, :]


def kernel(x):
    B, H, W, C = x.shape
    HO, WO = H // 2, W // 2
    grid = (B, HO // _KH)
    return pl.pallas_call(
        _pool_body,
        grid=grid,
        in_specs=[pl.BlockSpec((1, 2 * _KH, W, C), lambda b, i: (b, i, 0, 0))],
        out_specs=pl.BlockSpec((1, _KH, WO, C), lambda b, i: (b, i, 0, 0)),
        out_shape=jax.ShapeDtypeStruct((B, HO, WO, C), x.dtype),
    )(x)


# speedup vs baseline: 5.7712x; 1.9154x over previous
import jax
import jax.numpy as jnp
from jax import lax
from jax.experimental import pallas as pl
from jax.experimental.pallas import tpu as pltpu


_KH = 14


def _pool_body(x_ref, o_ref):
    KH, C, WO = o_ref.shape[1:]
    W = 2 * WO
    y = jnp.maximum(x_ref[0], 0.0)
    yr = y.reshape(KH, 2, C, W)
    t = yr[:, 0]
    b = yr[:, 1]
    r = t + b
    r2 = t * t + b * b
    u = lax.broadcasted_iota(jnp.int32, (W, WO), 0)
    j = lax.broadcasted_iota(jnp.int32, (W, WO), 1)
    q = jnp.where(u // 2 == j, 1.0, 0.0).astype(jnp.float32)
    dn = (((2,), (0,)), ((), ()))
    s = lax.dot_general(r, q, dn, precision=lax.Precision.HIGHEST,
                        preferred_element_type=jnp.float32)
    s2 = lax.dot_general(r2, q, dn, precision=lax.Precision.HIGHEST,
                         preferred_element_type=jnp.float32)
    o_ref[0] = jnp.where(s > 0, s2 / jnp.where(s > 0, s, 1.0), 0.0)


def kernel(x):
    B, H, W, C = x.shape
    HO, WO = H // 2, W // 2
    xt = jnp.transpose(x, (0, 1, 3, 2))
    grid = (B, HO // _KH)
    ot = pl.pallas_call(
        _pool_body,
        grid=grid,
        in_specs=[pl.BlockSpec((1, 2 * _KH, C, W), lambda b, i: (b, i, 0, 0))],
        out_specs=pl.BlockSpec((1, _KH, C, WO), lambda b, i: (b, i, 0, 0)),
        out_shape=jax.ShapeDtypeStruct((B, HO, C, WO), x.dtype),
    )(xt)
    return jnp.transpose(ot, (0, 1, 3, 2))


# speedup vs baseline: 11.8858x; 2.0595x over previous
import jax
import jax.numpy as jnp
from jax import lax
from jax.experimental import pallas as pl
from jax.experimental.pallas import tpu as pltpu


_KH = 28


def _pool_body(x_ref, o_ref):
    KH, C, WO = o_ref.shape[1:]
    W = 2 * WO
    y = jnp.maximum(x_ref[0], 0.0)
    yr = y.reshape(KH, 2, C, W)
    t = yr[:, 0]
    b = yr[:, 1]
    r = t + b
    r2 = t * t + b * b
    u = lax.broadcasted_iota(jnp.int32, (W, WO), 0)
    j = lax.broadcasted_iota(jnp.int32, (W, WO), 1)
    q = jnp.where(u // 2 == j, 1.0, 0.0).astype(jnp.bfloat16)
    dn = (((2,), (0,)), ((), ()))

    def pair_sum(a):
        hi = a.astype(jnp.bfloat16)
        lo = (a - hi.astype(jnp.float32)).astype(jnp.bfloat16)
        d = lambda m: lax.dot_general(m, q, dn,
                                      preferred_element_type=jnp.float32)
        return d(hi) + d(lo)

    s = pair_sum(r)
    s2 = pair_sum(r2)
    o_ref[0] = jnp.where(s > 0, s2 / jnp.where(s > 0, s, 1.0), 0.0)


def kernel(x):
    B, H, W, C = x.shape
    HO, WO = H // 2, W // 2
    xt = jnp.transpose(x, (0, 1, 3, 2))
    grid = (B, HO // _KH)
    ot = pl.pallas_call(
        _pool_body,
        grid=grid,
        in_specs=[pl.BlockSpec((1, 2 * _KH, C, W), lambda b, i: (b, i, 0, 0))],
        out_specs=pl.BlockSpec((1, _KH, C, WO), lambda b, i: (b, i, 0, 0)),
        out_shape=jax.ShapeDtypeStruct((B, HO, C, WO), x.dtype),
    )(xt)
    return jnp.transpose(ot, (0, 1, 3, 2))
